# Initial kernel scaffold; baseline (speedup 1.0000x reference)
import jax
import jax.numpy as jnp
from jax.experimental import pallas as pl


def kernel(x, edge_index, graph_ids, W_gnn, b_gnn, W_mlp, b_mlp):
    raise NotImplementedError("write your pallas kernel here")


# speedup vs baseline: 5.6803x; 5.6803x over previous
import functools

import jax
import jax.numpy as jnp
from jax import lax
from jax.experimental import pallas as pl
from jax.experimental.pallas import tpu as pltpu
from jax.experimental.pallas import tpu_sc as plsc

_N = 10000
_E = 320000
_D = 128
_G = 64
_C = 10

_NC = 2
_NS = 16
_NW = _NC * _NS
_EW = _E // _NW
_CH = 80
_NCHUNK = _EW // _CH
_NP = 10240
_RT = _NP // _NS

@functools.cache
def _build_edge_agg():
    mesh = plsc.VectorSubcoreMesh(core_axis_name="c", subcore_axis_name="s")
    return pl.kernel(
        _edge_agg_body,
        out_type=jax.ShapeDtypeStruct((_NC * _NP, _D), jnp.float32),
        mesh=mesh,
        scratch_types=[
            pltpu.VMEM((_CH,), jnp.int32),
            pltpu.VMEM((_CH,), jnp.int32),
            pltpu.VMEM((_CH, _D), jnp.float32),
            pltpu.VMEM_SHARED((_NP, _D), jnp.float32),
            pltpu.SemaphoreType.DMA,
        ],
    )


def _edge_agg_body(x_hbm, src_hbm, dst_hbm, zero_hbm, out_hbm,
                   src_v, dst_v, rows_v, acc_sh, sem):
    c = lax.axis_index("c")
    s = lax.axis_index("s")
    wid = s * _NC + c

    pltpu.sync_copy(zero_hbm, acc_sh.at[pl.ds(s * _RT, _RT)])
    plsc.subcore_barrier()

    def body(i, carry):
        base = wid * _EW + i * _CH
        pltpu.sync_copy(src_hbm.at[pl.ds(base, _CH)], src_v)
        pltpu.sync_copy(dst_hbm.at[pl.ds(base, _CH)], dst_v)
        pltpu.async_copy(x_hbm.at[src_v], rows_v, sem).wait()
        pltpu.sync_copy(rows_v, acc_sh.at[dst_v], add=True)
        return carry

    lax.fori_loop(0, _NCHUNK, body, 0)

    plsc.subcore_barrier()
    pltpu.sync_copy(acc_sh.at[pl.ds(s * _RT, _RT)],
                    out_hbm.at[pl.ds(c * _NP + s * _RT, _RT)])


_BLK = 1280
_NB = _NP // _BLK


def _dense_body(parts_ref, gid_ref, wg_ref, bg_ref, wm_ref, bm_ref,
                out_ref, gsum_ref):
    i = pl.program_id(0)
    agg = parts_ref[0] + parts_ref[1]
    nr = jnp.maximum(
        jnp.dot(agg, wg_ref[...], preferred_element_type=jnp.float32)
        + bg_ref[...], 0.0)
    gid = gid_ref[0, 0, :]
    onehot = (lax.broadcasted_iota(jnp.int32, (_G, _BLK), 0)
              == gid[None, :]).astype(jnp.float32)
    part = jnp.dot(onehot, nr, preferred_element_type=jnp.float32)

    @pl.when(i == 0)
    def _init():
        gsum_ref[...] = part

    @pl.when(i > 0)
    def _acc():
        gsum_ref[...] += part

    @pl.when(i == _NB - 1)
    def _fin():
        out_ref[...] = (jnp.dot(gsum_ref[...], wm_ref[...],
                                preferred_element_type=jnp.float32)
                        + bm_ref[...])


def _dense(parts, gids3, W_gnn, b_gnn, W_mlp, b_mlp):
    return pl.pallas_call(
        _dense_body,
        grid=(_NB,),
        in_specs=[
            pl.BlockSpec((2, _BLK, _D), lambda i: (0, i, 0)),
            pl.BlockSpec((1, 1, _BLK), lambda i: (i, 0, 0)),
            pl.BlockSpec((_D, _D), lambda i: (0, 0)),
            pl.BlockSpec((1, _D), lambda i: (0, 0)),
            pl.BlockSpec((_D, _C), lambda i: (0, 0)),
            pl.BlockSpec((1, _C), lambda i: (0, 0)),
        ],
        out_specs=pl.BlockSpec((_G, _C), lambda i: (0, 0)),
        out_shape=jax.ShapeDtypeStruct((_G, _C), jnp.float32),
        scratch_shapes=[pltpu.VMEM((_G, _D), jnp.float32)],
        compiler_params=pltpu.CompilerParams(
            dimension_semantics=("arbitrary",)),
    )(parts, gids3, W_gnn, b_gnn, W_mlp, b_mlp)


def kernel(x, edge_index, graph_ids, W_gnn, b_gnn, W_mlp, b_mlp):
    src = edge_index[0].astype(jnp.int32)
    dst = edge_index[1].astype(jnp.int32)
    zero = jnp.zeros((_RT, _D), jnp.float32)
    parts = _build_edge_agg()(x, src, dst, zero)
    parts = parts.reshape(_NC, _NP, _D)
    gids = jnp.concatenate([graph_ids.astype(jnp.int32),
                            jnp.full((_NP - _N,), -1, jnp.int32)])
    gids3 = gids.reshape(_NB, 1, _BLK)
    return _dense(parts, gids3, W_gnn,
                  b_gnn.reshape(1, _D), W_mlp, b_mlp.reshape(1, _C))
